# Initial kernel scaffold; baseline (speedup 1.0000x reference)
import jax
import jax.numpy as jnp
from jax.experimental import pallas as pl


def kernel(x, edge_index, nodes, W_enc, W_cls):
    raise NotImplementedError("write your pallas kernel here")


# speedup vs baseline: 3.5132x; 3.5132x over previous
import jax
import jax.numpy as jnp
from jax import lax
from jax.experimental import pallas as pl
from jax.experimental.pallas import tpu as pltpu
from jax.experimental.pallas import tpu_sc as plsc

F32 = jnp.float32
I32 = jnp.int32

N_NODES = 10000
D = 128
E = 320000
B = 10000
NC = 2
NS = 16
NW = NC * NS
CHUNK = 128
ROWS_PT = 80
E_PAD = NW * ROWS_PT * CHUNK
EROWS = E_PAD // CHUNK
NPAD = 10240
TRASH = NPAD - 1
ZROWS = 32
SLICE_PT = NPAD // NS
B_PAD = 10240
BROWS = B_PAD // CHUNK
NMP = 64
NPIECE = SLICE_PT // NMP


def _seg_body(x, src2d, dst2d, featcat,
              idx_s, idx_d, rows, zbuf, acc, sem):
  c = lax.axis_index("c")
  s = lax.axis_index("s")
  z16 = jnp.zeros((16,), F32)

  def zrow(i, carry):
    for k in range(8):
      zbuf[i, pl.ds(k * 16, 16)] = z16
    return carry

  lax.fori_loop(0, ZROWS, zrow, 0)
  for k in range(SLICE_PT // ZROWS):
    pltpu.sync_copy(zbuf, acc.at[pl.ds(s * SLICE_PT + k * ZROWS, ZROWS)])
  plsc.subcore_barrier()

  base = (c * NS + s) * ROWS_PT
  pltpu.sync_copy(src2d.at[pl.ds(base, ROWS_PT)], idx_s)
  pltpu.sync_copy(dst2d.at[pl.ds(base, ROWS_PT)], idx_d)

  def chunk(j, carry):
    pltpu.async_copy(x.at[idx_s.at[j]], rows, sem).wait()
    pltpu.sync_copy(rows, acc.at[idx_d.at[j]], add=True)
    return carry

  lax.fori_loop(0, ROWS_PT, chunk, 0)
  plsc.subcore_barrier()

  sl = pl.ds(s * SLICE_PT, SLICE_PT)
  pltpu.sync_copy(acc.at[sl],
                  featcat.at[pl.ds(c * NPAD + s * SLICE_PT, SLICE_PT)])


def _deg_body(dst2d, degcat, idx_d, ones, zbuf, dacc, sem):
  c = lax.axis_index("c")
  s = lax.axis_index("s")
  z16 = jnp.zeros((16,), F32)
  one16 = jnp.ones((16,), F32)

  def zrow(i, carry):
    for k in range(8):
      zbuf[i, pl.ds(k * 16, 16)] = z16
    return carry

  lax.fori_loop(0, ZROWS, zrow, 0)

  def orow(i, carry):
    for k in range(8):
      ones[i, pl.ds(k * 16, 16)] = one16
    return carry

  lax.fori_loop(0, CHUNK, orow, 0)
  for k in range(SLICE_PT // ZROWS):
    pltpu.sync_copy(zbuf, dacc.at[pl.ds(s * SLICE_PT + k * ZROWS, ZROWS)])
  plsc.subcore_barrier()

  base = (c * NS + s) * ROWS_PT
  pltpu.sync_copy(dst2d.at[pl.ds(base, ROWS_PT)], idx_d)

  def chunk(j, carry):
    pltpu.sync_copy(ones, dacc.at[idx_d.at[j]], add=True)
    return carry

  lax.fori_loop(0, ROWS_PT, chunk, 0)
  plsc.subcore_barrier()

  sl = pl.ds(s * SLICE_PT, SLICE_PT)
  pltpu.sync_copy(dacc.at[sl],
                  degcat.at[pl.ds(c * NPAD + s * SLICE_PT, SLICE_PT)])


def _gather_body(x, featcat, degcat, nodes2d, out_s, out_n, nmean,
                 nidx, xg, nmg, p0b, p1b, d0b, d1b, nmb, sem):
  c = lax.axis_index("c")
  s = lax.axis_index("s")
  wid = c * NS + s
  one16 = jnp.ones((16,), F32)

  for piece in range(NPIECE):
    rb = s * SLICE_PT + piece * NMP
    pltpu.sync_copy(featcat.at[pl.ds(rb, NMP)], p0b)
    pltpu.sync_copy(featcat.at[pl.ds(NPAD + rb, NMP)], p1b)
    pltpu.sync_copy(degcat.at[pl.ds(rb, NMP)], d0b)
    pltpu.sync_copy(degcat.at[pl.ds(NPAD + rb, NMP)], d1b)

    def rowf(r, carry):
      deg16 = d0b[r, pl.ds(0, 16)] + d1b[r, pl.ds(0, 16)]
      inv = one16 / jnp.maximum(deg16, one16)
      for cc in range(8):
        slc = pl.ds(cc * 16, 16)
        nmb[r, slc] = (p0b[r, slc] + p1b[r, slc]) * inv
      return carry

    lax.fori_loop(0, NMP, rowf, 0)
    pltpu.sync_copy(nmb, nmean.at[pl.ds(rb, NMP)])
  plsc.subcore_barrier()

  nb = (BROWS - wid + NW - 1) // NW

  def bchunk(k, carry):
    j = wid + k * NW
    pltpu.sync_copy(nodes2d.at[j], nidx)
    pltpu.async_copy(x.at[nidx], xg, sem).wait()
    pltpu.async_copy(nmean.at[nidx], nmg, sem).wait()
    pltpu.sync_copy(xg, out_s.at[j])
    pltpu.sync_copy(nmg, out_n.at[j])
    return carry

  lax.fori_loop(0, nb, bchunk, 0)


def _mm_body(s_ref, n_ref, wsT, wnT, wclsT, o_ref):
  h = jnp.dot(s_ref[...], wsT[...], preferred_element_type=F32)
  h = h + jnp.dot(n_ref[...], wnT[...], preferred_element_type=F32)
  h = jnp.maximum(h, 0.0)
  o_ref[...] = jnp.dot(h, wclsT[...], preferred_element_type=F32)


_SC_MESH = plsc.VectorSubcoreMesh(core_axis_name="c", subcore_axis_name="s")

_seg = pl.kernel(
    _seg_body,
    out_type=jax.ShapeDtypeStruct((NC * NPAD, D), F32),
    mesh=_SC_MESH,
    scratch_types=(
        pltpu.VMEM((ROWS_PT, CHUNK), I32),
        pltpu.VMEM((ROWS_PT, CHUNK), I32),
        pltpu.VMEM((CHUNK, D), F32),
        pltpu.VMEM((ZROWS, D), F32),
        pltpu.VMEM_SHARED((NPAD, D), F32),
        pltpu.SemaphoreType.DMA,
    ),
)

_deg = pl.kernel(
    _deg_body,
    out_type=jax.ShapeDtypeStruct((NC * NPAD, D), F32),
    mesh=_SC_MESH,
    scratch_types=(
        pltpu.VMEM((ROWS_PT, CHUNK), I32),
        pltpu.VMEM((CHUNK, D), F32),
        pltpu.VMEM((ZROWS, D), F32),
        pltpu.VMEM_SHARED((NPAD, D), F32),
        pltpu.SemaphoreType.DMA,
    ),
)

_gath = pl.kernel(
    _gather_body,
    out_type=(
        jax.ShapeDtypeStruct((BROWS, CHUNK, D), F32),
        jax.ShapeDtypeStruct((BROWS, CHUNK, D), F32),
        jax.ShapeDtypeStruct((NPAD, D), F32),
    ),
    mesh=_SC_MESH,
    scratch_types=(
        pltpu.VMEM((CHUNK,), I32),
        pltpu.VMEM((CHUNK, D), F32),
        pltpu.VMEM((CHUNK, D), F32),
        pltpu.VMEM((NMP, D), F32),
        pltpu.VMEM((NMP, D), F32),
        pltpu.VMEM((NMP, D), F32),
        pltpu.VMEM((NMP, D), F32),
        pltpu.VMEM((NMP, D), F32),
        pltpu.SemaphoreType.DMA,
    ),
)

_MB = 640


def _matmul(sf, nf, wsT, wnT, wclsT):
  return pl.pallas_call(
      _mm_body,
      grid=(B_PAD // _MB,),
      in_specs=[
          pl.BlockSpec((_MB, D), lambda i: (i, 0)),
          pl.BlockSpec((_MB, D), lambda i: (i, 0)),
          pl.BlockSpec((D, D), lambda i: (0, 0)),
          pl.BlockSpec((D, D), lambda i: (0, 0)),
          pl.BlockSpec((D, D), lambda i: (0, 0)),
      ],
      out_specs=pl.BlockSpec((_MB, D), lambda i: (i, 0)),
      out_shape=jax.ShapeDtypeStruct((B_PAD, D), F32),
  )(sf, nf, wsT, wnT, wclsT)


def kernel(x, edge_index, nodes, W_enc, W_cls):
  epad = E_PAD - E
  src = jnp.concatenate(
      [jnp.asarray(edge_index[0], I32), jnp.zeros((epad,), I32)])
  dstf = jnp.concatenate(
      [jnp.asarray(edge_index[1], I32), jnp.full((epad,), TRASH, I32)])
  src2d = src.reshape(EROWS, CHUNK)
  dst2d = dstf.reshape(EROWS, CHUNK)
  nd = jnp.concatenate(
      [jnp.asarray(nodes, I32), jnp.zeros((B_PAD - B,), I32)])
  nd2d = nd.reshape(BROWS, CHUNK)

  featcat = _seg(x, src2d, dst2d)
  degcat = _deg(dst2d)
  selfg, neighg, _ = _gath(x, featcat, degcat, nd2d)
  sf = selfg.reshape(B_PAD, D)
  nf = neighg.reshape(B_PAD, D)
  wsT = W_enc[:, :D].T
  wnT = W_enc[:, D:].T
  wclsT = W_cls.T
  return _matmul(sf, nf, wsT, wnT, wclsT)[:B]


# speedup vs baseline: 3.8000x; 1.0816x over previous
import jax
import jax.numpy as jnp
from jax import lax
from jax.experimental import pallas as pl
from jax.experimental.pallas import tpu as pltpu
from jax.experimental.pallas import tpu_sc as plsc

F32 = jnp.float32
I32 = jnp.int32

N_NODES = 10000
D = 128
E = 320000
B = 10000
NC = 2
NS = 16
NW = NC * NS
CHUNK = 128
ROWS_PT = 80
E_PAD = NW * ROWS_PT * CHUNK
EROWS = E_PAD // CHUNK
NPAD = 10240
TRASH = NPAD - 1
ZROWS = 32
SLICE_PT = NPAD // NS
B_PAD = 10240
BROWS = B_PAD // CHUNK
NMP = 64
NPIECE = SLICE_PT // NMP


def _seg_body(x, src2d, dst2d, featcat, degcat,
              idx_s, idx_d, rows0, rows1, zbuf, acc, sem0, sem1):
  c = lax.axis_index("c")
  s = lax.axis_index("s")
  z16 = jnp.zeros((16,), F32)

  def zrow(i, carry):
    for k in range(8):
      zbuf[i, pl.ds(k * 16, 16)] = z16
    return carry

  lax.fori_loop(0, ZROWS, zrow, 0)
  for k in range(SLICE_PT // ZROWS):
    pltpu.sync_copy(zbuf, acc.at[pl.ds(s * SLICE_PT + k * ZROWS, ZROWS)])
  plsc.subcore_barrier()

  base = (c * NS + s) * ROWS_PT
  HR = ROWS_PT // 2
  for half in range(2):
    pltpu.sync_copy(src2d.at[pl.ds(base + half * HR, HR)], idx_s)
    pltpu.sync_copy(dst2d.at[pl.ds(base + half * HR, HR)], idx_d)
    pltpu.async_copy(x.at[idx_s.at[0]], rows0, sem0)
    last = HR - 1

    def pair(jj, carry):
      j0 = jj * 2
      j1 = j0 + 1
      jn = jnp.minimum(j0 + 2, last)
      pltpu.make_async_copy(x.at[idx_s.at[j0]], rows0, sem0).wait()
      pltpu.async_copy(x.at[idx_s.at[j1]], rows1, sem1)
      pltpu.sync_copy(rows0, acc.at[idx_d.at[j0]], add=True)
      pltpu.make_async_copy(x.at[idx_s.at[j1]], rows1, sem1).wait()
      pltpu.async_copy(x.at[idx_s.at[jn]], rows0, sem0)
      pltpu.sync_copy(rows1, acc.at[idx_d.at[j1]], add=True)
      return carry

    lax.fori_loop(0, HR // 2, pair, 0)
    pltpu.make_async_copy(x.at[idx_s.at[last]], rows0, sem0).wait()
  plsc.subcore_barrier()

  sl = pl.ds(s * SLICE_PT, SLICE_PT)
  pltpu.sync_copy(acc.at[sl],
                  featcat.at[pl.ds(c * NPAD + s * SLICE_PT, SLICE_PT)])
  plsc.subcore_barrier()

  for k in range(SLICE_PT // ZROWS):
    pltpu.sync_copy(zbuf, acc.at[pl.ds(s * SLICE_PT + k * ZROWS, ZROWS)])
  one16 = jnp.ones((16,), F32)

  def orow(i, carry):
    for k in range(8):
      rows0[i, pl.ds(k * 16, 16)] = one16
    return carry

  lax.fori_loop(0, CHUNK, orow, 0)
  plsc.subcore_barrier()
  HR2 = ROWS_PT // 2
  for half in range(2):
    pltpu.sync_copy(dst2d.at[pl.ds(base + half * HR2, HR2)], idx_d)

    def dchunk(j, carry):
      pltpu.sync_copy(rows0, acc.at[idx_d.at[j]], add=True)
      return carry

    lax.fori_loop(0, HR2, dchunk, 0)
  plsc.subcore_barrier()
  pltpu.sync_copy(acc.at[sl],
                  degcat.at[pl.ds(c * NPAD + s * SLICE_PT, SLICE_PT)])


def _gather_body(x, featcat, degcat, nodes2d, out_s, out_n, nmean,
                 nidx, xg, nmg, p0b, p1b, d0b, d1b, nmb, sem):
  c = lax.axis_index("c")
  s = lax.axis_index("s")
  wid = c * NS + s
  one16 = jnp.ones((16,), F32)

  for piece in range(NPIECE):
    rb = s * SLICE_PT + piece * NMP
    pltpu.async_copy(featcat.at[pl.ds(rb, NMP)], p0b, sem)
    pltpu.async_copy(featcat.at[pl.ds(NPAD + rb, NMP)], p1b, sem)
    pltpu.async_copy(degcat.at[pl.ds(rb, NMP)], d0b, sem)
    pltpu.async_copy(degcat.at[pl.ds(NPAD + rb, NMP)], d1b, sem)
    pltpu.make_async_copy(featcat.at[pl.ds(rb, NMP)], p0b, sem).wait()
    pltpu.make_async_copy(featcat.at[pl.ds(NPAD + rb, NMP)], p1b, sem).wait()
    pltpu.make_async_copy(degcat.at[pl.ds(rb, NMP)], d0b, sem).wait()
    pltpu.make_async_copy(degcat.at[pl.ds(NPAD + rb, NMP)], d1b, sem).wait()

    def rowf(r, carry):
      deg16 = d0b[r, pl.ds(0, 16)] + d1b[r, pl.ds(0, 16)]
      inv = one16 / jnp.maximum(deg16, one16)
      for cc in range(8):
        slc = pl.ds(cc * 16, 16)
        nmb[r, slc] = (p0b[r, slc] + p1b[r, slc]) * inv
      return carry

    lax.fori_loop(0, NMP, rowf, 0)
    pltpu.sync_copy(nmb, nmean.at[pl.ds(rb, NMP)])
  plsc.subcore_barrier()

  nb = (BROWS - wid + NW - 1) // NW

  def bchunk(k, carry):
    j = wid + k * NW
    pltpu.sync_copy(nodes2d.at[j], nidx)
    pltpu.async_copy(x.at[nidx], xg, sem)
    pltpu.async_copy(nmean.at[nidx], nmg, sem)
    pltpu.make_async_copy(x.at[nidx], xg, sem).wait()
    pltpu.make_async_copy(nmean.at[nidx], nmg, sem).wait()
    pltpu.sync_copy(xg, out_s.at[j])
    pltpu.sync_copy(nmg, out_n.at[j])
    return carry

  lax.fori_loop(0, nb, bchunk, 0)


def _mm_body(s_ref, n_ref, wsT, wnT, wclsT, o_ref):
  h = jnp.dot(s_ref[...], wsT[...], preferred_element_type=F32)
  h = h + jnp.dot(n_ref[...], wnT[...], preferred_element_type=F32)
  h = jnp.maximum(h, 0.0)
  o_ref[...] = jnp.dot(h, wclsT[...], preferred_element_type=F32)


_SC_MESH = plsc.VectorSubcoreMesh(core_axis_name="c", subcore_axis_name="s")

_seg = pl.kernel(
    _seg_body,
    out_type=(jax.ShapeDtypeStruct((NC * NPAD, D), F32),
              jax.ShapeDtypeStruct((NC * NPAD, D), F32)),
    mesh=_SC_MESH,
    scratch_types=(
        pltpu.VMEM((ROWS_PT // 2, CHUNK), I32),
        pltpu.VMEM((ROWS_PT // 2, CHUNK), I32),
        pltpu.VMEM((CHUNK, D), F32),
        pltpu.VMEM((CHUNK, D), F32),
        pltpu.VMEM((ZROWS, D), F32),
        pltpu.VMEM_SHARED((NPAD, D), F32),
        pltpu.SemaphoreType.DMA,
        pltpu.SemaphoreType.DMA,
    ),
)

_gath = pl.kernel(
    _gather_body,
    out_type=(
        jax.ShapeDtypeStruct((BROWS, CHUNK, D), F32),
        jax.ShapeDtypeStruct((BROWS, CHUNK, D), F32),
        jax.ShapeDtypeStruct((NPAD, D), F32),
    ),
    mesh=_SC_MESH,
    scratch_types=(
        pltpu.VMEM((CHUNK,), I32),
        pltpu.VMEM((CHUNK, D), F32),
        pltpu.VMEM((CHUNK, D), F32),
        pltpu.VMEM((NMP, D), F32),
        pltpu.VMEM((NMP, D), F32),
        pltpu.VMEM((NMP, D), F32),
        pltpu.VMEM((NMP, D), F32),
        pltpu.VMEM((NMP, D), F32),
        pltpu.SemaphoreType.DMA,
    ),
)

_MB = 640


def _matmul(sf, nf, wsT, wnT, wclsT):
  return pl.pallas_call(
      _mm_body,
      grid=(B_PAD // _MB,),
      in_specs=[
          pl.BlockSpec((_MB, D), lambda i: (i, 0)),
          pl.BlockSpec((_MB, D), lambda i: (i, 0)),
          pl.BlockSpec((D, D), lambda i: (0, 0)),
          pl.BlockSpec((D, D), lambda i: (0, 0)),
          pl.BlockSpec((D, D), lambda i: (0, 0)),
      ],
      out_specs=pl.BlockSpec((_MB, D), lambda i: (i, 0)),
      out_shape=jax.ShapeDtypeStruct((B_PAD, D), F32),
  )(sf, nf, wsT, wnT, wclsT)


def kernel(x, edge_index, nodes, W_enc, W_cls):
  epad = E_PAD - E
  src = jnp.concatenate(
      [jnp.asarray(edge_index[0], I32), jnp.zeros((epad,), I32)])
  dstf = jnp.concatenate(
      [jnp.asarray(edge_index[1], I32), jnp.full((epad,), TRASH, I32)])
  src2d = src.reshape(EROWS, CHUNK)
  dst2d = dstf.reshape(EROWS, CHUNK)
  nd = jnp.concatenate(
      [jnp.asarray(nodes, I32), jnp.zeros((B_PAD - B,), I32)])
  nd2d = nd.reshape(BROWS, CHUNK)

  featcat, degcat = _seg(x, src2d, dst2d)
  selfg, neighg, _ = _gath(x, featcat, degcat, nd2d)
  sf = selfg.reshape(B_PAD, D)
  nf = neighg.reshape(B_PAD, D)
  wsT = W_enc[:, :D].T
  wnT = W_enc[:, D:].T
  wclsT = W_cls.T
  return _matmul(sf, nf, wsT, wnT, wclsT)[:B]


# speedup vs baseline: 3.8991x; 1.0261x over previous
import jax
import jax.numpy as jnp
from jax import lax
from jax.experimental import pallas as pl
from jax.experimental.pallas import tpu as pltpu
from jax.experimental.pallas import tpu_sc as plsc

F32 = jnp.float32
I32 = jnp.int32

N_NODES = 10000
D = 128
E = 320000
B = 10000
NC = 2
NS = 16
NW = NC * NS
CHUNK = 128
ROWS_PT = 80
E_PAD = NW * ROWS_PT * CHUNK
EROWS = E_PAD // CHUNK
NPAD = 10240
TRASH = NPAD - 1
ZROWS = 32
SLICE_PT = NPAD // NS
B_PAD = 10240
BROWS = B_PAD // CHUNK
NMP = 64
NPIECE = SLICE_PT // NMP


def _seg_body(x, src2d, dst2d, featcat, degcat,
              idx_s, idx_d, rows0, rows1, zbuf, acc, sem0, sem1):
  c = lax.axis_index("c")
  s = lax.axis_index("s")
  z16 = jnp.zeros((16,), F32)

  def zrow(i, carry):
    for k in range(8):
      zbuf[i, pl.ds(k * 16, 16)] = z16
    return carry

  lax.fori_loop(0, ZROWS, zrow, 0)
  for k in range(SLICE_PT // ZROWS):
    pltpu.sync_copy(zbuf, acc.at[pl.ds(s * SLICE_PT + k * ZROWS, ZROWS)])
  plsc.subcore_barrier()

  base = (c * NS + s) * ROWS_PT
  HR = ROWS_PT // 2
  for half in range(2):
    pltpu.sync_copy(src2d.at[pl.ds(base + half * HR, HR)], idx_s)
    pltpu.sync_copy(dst2d.at[pl.ds(base + half * HR, HR)], idx_d)
    pltpu.async_copy(x.at[idx_s.at[0]], rows0, sem0)
    last = HR - 1

    def pair(jj, carry):
      j0 = jj * 2
      j1 = j0 + 1
      jn = jnp.minimum(j0 + 2, last)
      pltpu.make_async_copy(x.at[idx_s.at[j0]], rows0, sem0).wait()
      pltpu.async_copy(x.at[idx_s.at[j1]], rows1, sem1)
      pltpu.sync_copy(rows0, acc.at[idx_d.at[j0]], add=True)
      pltpu.make_async_copy(x.at[idx_s.at[j1]], rows1, sem1).wait()
      pltpu.async_copy(x.at[idx_s.at[jn]], rows0, sem0)
      pltpu.sync_copy(rows1, acc.at[idx_d.at[j1]], add=True)
      return carry

    lax.fori_loop(0, HR // 2, pair, 0)
    pltpu.make_async_copy(x.at[idx_s.at[last]], rows0, sem0).wait()
  plsc.subcore_barrier()

  sl = pl.ds(s * SLICE_PT, SLICE_PT)
  pltpu.sync_copy(acc.at[sl],
                  featcat.at[pl.ds(c * NPAD + s * SLICE_PT, SLICE_PT)])
  plsc.subcore_barrier()

  for k in range(SLICE_PT // ZROWS):
    pltpu.sync_copy(zbuf, acc.at[pl.ds(s * SLICE_PT + k * ZROWS, ZROWS)])
  one16 = jnp.ones((16,), F32)

  def orow(i, carry):
    for k in range(8):
      rows0[i, pl.ds(k * 16, 16)] = one16
    return carry

  lax.fori_loop(0, CHUNK, orow, 0)
  plsc.subcore_barrier()
  HR2 = ROWS_PT // 2
  for half in range(2):
    pltpu.sync_copy(dst2d.at[pl.ds(base + half * HR2, HR2)], idx_d)

    def oct(kk, carry):
      for b in range(8):
        pltpu.async_copy(rows0, acc.at[idx_d.at[kk * 8 + b]], sem0,
                         add=True)
      for b in range(8):
        pltpu.make_async_copy(rows0, acc.at[idx_d.at[kk * 8 + b]],
                              sem0).wait()
      return carry

    lax.fori_loop(0, HR2 // 8, oct, 0)
  plsc.subcore_barrier()
  pltpu.sync_copy(acc.at[sl],
                  degcat.at[pl.ds(c * NPAD + s * SLICE_PT, SLICE_PT)])


def _gather_body(x, featcat, degcat, nodes2d, out_s, out_n, nmean,
                 nidx, xg, nmg, p0b, p1b, d0b, d1b, nmb, sem):
  c = lax.axis_index("c")
  s = lax.axis_index("s")
  wid = c * NS + s
  one16 = jnp.ones((16,), F32)

  for piece in range(NPIECE):
    rb = s * SLICE_PT + piece * NMP
    pltpu.async_copy(featcat.at[pl.ds(rb, NMP)], p0b, sem)
    pltpu.async_copy(featcat.at[pl.ds(NPAD + rb, NMP)], p1b, sem)
    pltpu.async_copy(degcat.at[pl.ds(rb, NMP)], d0b, sem)
    pltpu.async_copy(degcat.at[pl.ds(NPAD + rb, NMP)], d1b, sem)
    pltpu.make_async_copy(featcat.at[pl.ds(rb, NMP)], p0b, sem).wait()
    pltpu.make_async_copy(featcat.at[pl.ds(NPAD + rb, NMP)], p1b, sem).wait()
    pltpu.make_async_copy(degcat.at[pl.ds(rb, NMP)], d0b, sem).wait()
    pltpu.make_async_copy(degcat.at[pl.ds(NPAD + rb, NMP)], d1b, sem).wait()

    def rowf(r, carry):
      deg16 = d0b[r, pl.ds(0, 16)] + d1b[r, pl.ds(0, 16)]
      inv = one16 / jnp.maximum(deg16, one16)
      for cc in range(8):
        slc = pl.ds(cc * 16, 16)
        nmb[r, slc] = (p0b[r, slc] + p1b[r, slc]) * inv
      return carry

    lax.fori_loop(0, NMP, rowf, 0)
    pltpu.sync_copy(nmb, nmean.at[pl.ds(rb, NMP)])
  plsc.subcore_barrier()

  nb = (BROWS - wid + NW - 1) // NW

  def bchunk(k, carry):
    j = wid + k * NW
    pltpu.sync_copy(nodes2d.at[j], nidx)
    pltpu.async_copy(x.at[nidx], xg, sem)
    pltpu.async_copy(nmean.at[nidx], nmg, sem)
    pltpu.make_async_copy(x.at[nidx], xg, sem).wait()
    pltpu.make_async_copy(nmean.at[nidx], nmg, sem).wait()
    pltpu.sync_copy(xg, out_s.at[j])
    pltpu.sync_copy(nmg, out_n.at[j])
    return carry

  lax.fori_loop(0, nb, bchunk, 0)


def _mm_body(s_ref, n_ref, wsT, wnT, wclsT, o_ref):
  h = jnp.dot(s_ref[...], wsT[...], preferred_element_type=F32)
  h = h + jnp.dot(n_ref[...], wnT[...], preferred_element_type=F32)
  h = jnp.maximum(h, 0.0)
  o_ref[...] = jnp.dot(h, wclsT[...], preferred_element_type=F32)


_SC_MESH = plsc.VectorSubcoreMesh(core_axis_name="c", subcore_axis_name="s")

_seg = pl.kernel(
    _seg_body,
    out_type=(jax.ShapeDtypeStruct((NC * NPAD, D), F32),
              jax.ShapeDtypeStruct((NC * NPAD, D), F32)),
    mesh=_SC_MESH,
    scratch_types=(
        pltpu.VMEM((ROWS_PT // 2, CHUNK), I32),
        pltpu.VMEM((ROWS_PT // 2, CHUNK), I32),
        pltpu.VMEM((CHUNK, D), F32),
        pltpu.VMEM((CHUNK, D), F32),
        pltpu.VMEM((ZROWS, D), F32),
        pltpu.VMEM_SHARED((NPAD, D), F32),
        pltpu.SemaphoreType.DMA,
        pltpu.SemaphoreType.DMA,
    ),
)

_gath = pl.kernel(
    _gather_body,
    out_type=(
        jax.ShapeDtypeStruct((BROWS, CHUNK, D), F32),
        jax.ShapeDtypeStruct((BROWS, CHUNK, D), F32),
        jax.ShapeDtypeStruct((NPAD, D), F32),
    ),
    mesh=_SC_MESH,
    scratch_types=(
        pltpu.VMEM((CHUNK,), I32),
        pltpu.VMEM((CHUNK, D), F32),
        pltpu.VMEM((CHUNK, D), F32),
        pltpu.VMEM((NMP, D), F32),
        pltpu.VMEM((NMP, D), F32),
        pltpu.VMEM((NMP, D), F32),
        pltpu.VMEM((NMP, D), F32),
        pltpu.VMEM((NMP, D), F32),
        pltpu.SemaphoreType.DMA,
    ),
)

_MB = 640


def _matmul(sf, nf, wsT, wnT, wclsT):
  return pl.pallas_call(
      _mm_body,
      grid=(B_PAD // _MB,),
      in_specs=[
          pl.BlockSpec((_MB, D), lambda i: (i, 0)),
          pl.BlockSpec((_MB, D), lambda i: (i, 0)),
          pl.BlockSpec((D, D), lambda i: (0, 0)),
          pl.BlockSpec((D, D), lambda i: (0, 0)),
          pl.BlockSpec((D, D), lambda i: (0, 0)),
      ],
      out_specs=pl.BlockSpec((_MB, D), lambda i: (i, 0)),
      out_shape=jax.ShapeDtypeStruct((B_PAD, D), F32),
  )(sf, nf, wsT, wnT, wclsT)


def kernel(x, edge_index, nodes, W_enc, W_cls):
  epad = E_PAD - E
  src = jnp.concatenate(
      [jnp.asarray(edge_index[0], I32), jnp.zeros((epad,), I32)])
  dstf = jnp.concatenate(
      [jnp.asarray(edge_index[1], I32), jnp.full((epad,), TRASH, I32)])
  src2d = src.reshape(EROWS, CHUNK)
  dst2d = dstf.reshape(EROWS, CHUNK)
  nd = jnp.concatenate(
      [jnp.asarray(nodes, I32), jnp.zeros((B_PAD - B,), I32)])
  nd2d = nd.reshape(BROWS, CHUNK)

  featcat, degcat = _seg(x, src2d, dst2d)
  selfg, neighg, _ = _gath(x, featcat, degcat, nd2d)
  sf = selfg.reshape(B_PAD, D)
  nf = neighg.reshape(B_PAD, D)
  wsT = W_enc[:, :D].T
  wnT = W_enc[:, D:].T
  wclsT = W_cls.T
  return _matmul(sf, nf, wsT, wnT, wclsT)[:B]
